# baseline (speedup 1.0000x reference)
import functools

import jax
import jax.numpy as jnp
from jax import lax
from jax.experimental import pallas as pl
from jax.experimental.pallas import tpu as pltpu
from jax.experimental.pallas import tpu_sc as plsc

_EMBED_DIM = 32

_info = plsc.get_sparse_core_info()
_NC, _NS = _info.num_cores, _info.num_subcores
_NW = _NC * _NS

_CHUNK = 800
_NBUF = 2


def _gather_kernel(n_flat, n_chunks):
    mesh = plsc.VectorSubcoreMesh(core_axis_name="c", subcore_axis_name="s")
    b_per_w = n_flat // _NW

    @functools.partial(
        pl.kernel,
        out_type=jax.ShapeDtypeStruct((n_flat, _EMBED_DIM), jnp.float32),
        mesh=mesh,
        scratch_types=[
            pltpu.VMEM((_NBUF, _CHUNK), jnp.int32),
            pltpu.VMEM_SHARED((_NS * _NC, _CHUNK), jnp.int32),
            pltpu.SMEM((_CHUNK,), jnp.int32),
            pltpu.VMEM((_NBUF, _CHUNK, _EMBED_DIM), jnp.float32),
            [pltpu.SemaphoreType.DMA] * _NBUF,
            [pltpu.SemaphoreType.DMA] * _NBUF,
            [pltpu.SemaphoreType.DMA] * _NBUF,
        ],
        compiler_params=pltpu.CompilerParams(use_tc_tiling_on_sc=False),
    )
    def k(idx_hbm, table_hbm, out_hbm, idx_v, idx_sh, idx_s, rows_v,
          idx_sems, g_sems, w_sems):
        wid = lax.axis_index("s") * _NC + lax.axis_index("c")
        sid = lax.axis_index("s")
        base = wid * b_per_w

        def off(i):
            return pl.multiple_of(base + i * _CHUNK, _CHUNK)

        for i in range(min(_NBUF, n_chunks)):
            pltpu.async_copy(idx_hbm.at[pl.ds(off(i), _CHUNK)], idx_v.at[i],
                             idx_sems[i])

        for i in range(n_chunks):
            b = i % _NBUF
            pltpu.make_async_copy(idx_hbm.at[pl.ds(off(i), _CHUNK)],
                                  idx_v.at[b], idx_sems[b]).wait()
            pltpu.sync_copy(idx_v.at[b], idx_sh.at[sid])
            pltpu.sync_copy(idx_sh.at[sid], idx_s)
            if i >= _NBUF:
                pltpu.make_async_copy(rows_v.at[b],
                                      out_hbm.at[pl.ds(off(i), _CHUNK)],
                                      w_sems[b]).wait()

            def row(j, carry):
                r = idx_s[j]
                pltpu.async_copy(table_hbm.at[pl.ds(r, 1)],
                                 rows_v.at[b, pl.ds(j, 1)], g_sems[b])
                return carry

            lax.fori_loop(0, _CHUNK, row, 0)
            pltpu.make_async_copy(table_hbm.at[pl.ds(0, _CHUNK)],
                                  rows_v.at[b], g_sems[b]).wait()

            if i + _NBUF < n_chunks:
                pltpu.async_copy(idx_hbm.at[pl.ds(off(i + _NBUF), _CHUNK)],
                                 idx_v.at[b], idx_sems[b])
            pltpu.async_copy(rows_v.at[b], out_hbm.at[pl.ds(off(i), _CHUNK)],
                             w_sems[b])

        for i in range(max(0, n_chunks - _NBUF), n_chunks):
            b = i % _NBUF
            pltpu.make_async_copy(rows_v.at[b],
                                  out_hbm.at[pl.ds(off(i), _CHUNK)],
                                  w_sems[b]).wait()

    return k


def kernel(card_indices, table):
    batch, hist = card_indices.shape
    n_flat = batch * hist
    idx_flat = card_indices.reshape(n_flat).astype(jnp.int32)
    n_chunks = n_flat // (_NW * _CHUNK)
    out = _gather_kernel(n_flat, n_chunks)(idx_flat, table)
    return out.reshape(batch, hist, _EMBED_DIM)


# speedup vs baseline: 1.0028x; 1.0028x over previous
import functools

import jax
import jax.numpy as jnp
from jax import lax
from jax.experimental import pallas as pl
from jax.experimental.pallas import tpu as pltpu
from jax.experimental.pallas import tpu_sc as plsc

_EMBED_DIM = 32

_info = plsc.get_sparse_core_info()
_NC, _NS = _info.num_cores, _info.num_subcores
_NW = _NC * _NS

_CHUNK = 1600
_DMA = 800
_DBATCH = 200
_STREAM = _CHUNK - _DMA
_NBUF = 2


def _gather_kernel(n_flat, n_chunks):
    mesh = plsc.VectorSubcoreMesh(core_axis_name="c", subcore_axis_name="s")
    b_per_w = n_flat // _NW

    @functools.partial(
        pl.kernel,
        out_type=jax.ShapeDtypeStruct((n_flat, _EMBED_DIM), jnp.float32),
        mesh=mesh,
        scratch_types=[
            pltpu.VMEM((_NBUF, _CHUNK), jnp.int32),
            pltpu.VMEM_SHARED((_NS * _NC, _DMA), jnp.int32),
            pltpu.SMEM((_DMA,), jnp.int32),
            pltpu.VMEM((_NBUF, _DMA, _EMBED_DIM), jnp.float32),
            pltpu.VMEM((_NBUF, _STREAM, _EMBED_DIM), jnp.float32),
            [pltpu.SemaphoreType.DMA] * _NBUF,
            [pltpu.SemaphoreType.DMA] * _NBUF,
            [pltpu.SemaphoreType.DMA] * _NBUF,
            [pltpu.SemaphoreType.DMA] * _NBUF,
            [pltpu.SemaphoreType.DMA] * _NBUF,
        ],
        compiler_params=pltpu.CompilerParams(use_tc_tiling_on_sc=False),
    )
    def k(idx_hbm, table_hbm, out_hbm, idx_v, idx_sh, idx_s, rows_d, rows_s,
          idx_sems, g_sems, d_sems, wd_sems, ws_sems):
        wid = lax.axis_index("s") * _NC + lax.axis_index("c")
        sid = lax.axis_index("s")
        base = wid * b_per_w

        def off(i):
            return pl.multiple_of(base + i * _CHUNK, _CHUNK)

        for i in range(min(_NBUF, n_chunks)):
            pltpu.async_copy(idx_hbm.at[pl.ds(off(i), _CHUNK)], idx_v.at[i],
                             idx_sems[i])

        for i in range(n_chunks):
            b = i % _NBUF
            pltpu.make_async_copy(idx_hbm.at[pl.ds(off(i), _CHUNK)],
                                  idx_v.at[b], idx_sems[b]).wait()
            pltpu.sync_copy(idx_v.at[b, pl.ds(0, _DMA)], idx_sh.at[sid])
            pltpu.sync_copy(idx_sh.at[sid], idx_s)
            if i >= _NBUF:
                pltpu.make_async_copy(rows_d.at[b],
                                      out_hbm.at[pl.ds(off(i), _DMA)],
                                      wd_sems[b]).wait()
                pltpu.make_async_copy(rows_s.at[b],
                                      out_hbm.at[pl.ds(off(i), _STREAM)],
                                      ws_sems[b]).wait()
            pltpu.async_copy(table_hbm.at[idx_v.at[b, pl.ds(_DMA, _STREAM)]],
                             rows_s.at[b], g_sems[b])

            for jb in range(0, _DMA, _DBATCH):
                def row(j, carry):
                    r = idx_s[j]
                    pltpu.async_copy(table_hbm.at[pl.ds(r, 1)],
                                     rows_d.at[b, pl.ds(j, 1)], d_sems[b])
                    return carry

                lax.fori_loop(jb, jb + _DBATCH, row, 0)
                pltpu.make_async_copy(
                    table_hbm.at[pl.ds(0, _DBATCH)],
                    rows_d.at[b, pl.ds(jb, _DBATCH)], d_sems[b]).wait()

            pltpu.make_async_copy(table_hbm.at[idx_v.at[b, pl.ds(_DMA, _STREAM)]],
                                  rows_s.at[b], g_sems[b]).wait()

            if i + _NBUF < n_chunks:
                pltpu.async_copy(idx_hbm.at[pl.ds(off(i + _NBUF), _CHUNK)],
                                 idx_v.at[b], idx_sems[b])
            pltpu.async_copy(rows_d.at[b],
                             out_hbm.at[pl.ds(off(i), _DMA)], wd_sems[b])
            pltpu.async_copy(rows_s.at[b],
                             out_hbm.at[pl.ds(pl.multiple_of(off(i) + _DMA, _DMA), _STREAM)],
                             ws_sems[b])

        for i in range(max(0, n_chunks - _NBUF), n_chunks):
            b = i % _NBUF
            pltpu.make_async_copy(rows_d.at[b],
                                  out_hbm.at[pl.ds(off(i), _DMA)],
                                  wd_sems[b]).wait()
            pltpu.make_async_copy(rows_s.at[b],
                                  out_hbm.at[pl.ds(off(i), _STREAM)],
                                  ws_sems[b]).wait()

    return k


def kernel(card_indices, table):
    batch, hist = card_indices.shape
    n_flat = batch * hist
    idx_flat = card_indices.reshape(n_flat).astype(jnp.int32)
    n_chunks = n_flat // (_NW * _CHUNK)
    out = _gather_kernel(n_flat, n_chunks)(idx_flat, table)
    return out.reshape(batch, hist, _EMBED_DIM)


# speedup vs baseline: 1.0389x; 1.0359x over previous
import functools

import jax
import jax.numpy as jnp
from jax import lax
from jax.experimental import pallas as pl
from jax.experimental.pallas import tpu as pltpu
from jax.experimental.pallas import tpu_sc as plsc

_EMBED_DIM = 32

_info = plsc.get_sparse_core_info()
_NC, _NS = _info.num_cores, _info.num_subcores
_NW = _NC * _NS

_CHUNK = 1600
_NBUF = 2


def _gather_kernel(n_flat, n_chunks):
    mesh = plsc.VectorSubcoreMesh(core_axis_name="c", subcore_axis_name="s")
    b_per_w = n_flat // _NW

    @functools.partial(
        pl.kernel,
        out_type=jax.ShapeDtypeStruct((n_flat, _EMBED_DIM), jnp.float32),
        mesh=mesh,
        scratch_types=[
            pltpu.VMEM((_NBUF, _CHUNK), jnp.int32),
            pltpu.VMEM((_NBUF, _CHUNK, _EMBED_DIM), jnp.float32),
            [pltpu.SemaphoreType.DMA] * _NBUF,
            [pltpu.SemaphoreType.DMA] * _NBUF,
            [pltpu.SemaphoreType.DMA] * _NBUF,
        ],
        compiler_params=pltpu.CompilerParams(use_tc_tiling_on_sc=False),
    )
    def k(idx_hbm, table_hbm, out_hbm, idx_v, rows_v, idx_sems, g_sems,
          w_sems):
        wid = lax.axis_index("s") * _NC + lax.axis_index("c")
        base = wid * b_per_w

        def off(i):
            return pl.multiple_of(base + i * _CHUNK, _CHUNK)

        def fire_gather(i):
            b = i % _NBUF
            pltpu.async_copy(table_hbm.at[idx_v.at[b]], rows_v.at[b],
                             g_sems[b])

        def wait_gather(i):
            b = i % _NBUF
            pltpu.make_async_copy(table_hbm.at[idx_v.at[b]], rows_v.at[b],
                                  g_sems[b]).wait()

        def fire_wb(i):
            b = i % _NBUF
            pltpu.async_copy(rows_v.at[b], out_hbm.at[pl.ds(off(i), _CHUNK)],
                             w_sems[b])

        def wait_wb(i):
            b = i % _NBUF
            pltpu.make_async_copy(rows_v.at[b],
                                  out_hbm.at[pl.ds(off(i), _CHUNK)],
                                  w_sems[b]).wait()

        def fire_idx(i):
            b = i % _NBUF
            pltpu.async_copy(idx_hbm.at[pl.ds(off(i), _CHUNK)], idx_v.at[b],
                             idx_sems[b])

        def wait_idx(i):
            b = i % _NBUF
            pltpu.make_async_copy(idx_hbm.at[pl.ds(off(i), _CHUNK)],
                                  idx_v.at[b], idx_sems[b]).wait()

        fire_idx(0)
        for i in range(n_chunks):
            wait_idx(i)
            fire_gather(i)
            if i > 0:
                wait_gather(i - 1)
                fire_wb(i - 1)
                if i + 1 < n_chunks:
                    fire_idx(i + 1)
                if i + 1 < n_chunks:
                    wait_wb(i - 1)
            elif n_chunks > 1:
                fire_idx(1)
        wait_gather(n_chunks - 1)
        fire_wb(n_chunks - 1)
        if n_chunks > 1:
            wait_wb(n_chunks - 2)
        wait_wb(n_chunks - 1)

    return k


def kernel(card_indices, table):
    batch, hist = card_indices.shape
    n_flat = batch * hist
    idx_flat = card_indices.reshape(n_flat).astype(jnp.int32)
    n_chunks = n_flat // (_NW * _CHUNK)
    out = _gather_kernel(n_flat, n_chunks)(idx_flat, table)
    return out.reshape(batch, hist, _EMBED_DIM)
